# baseline (speedup 1.0000x reference)
import functools

import jax
import jax.numpy as jnp
from jax import lax
from jax.experimental import pallas as pl
from jax.experimental.pallas import tpu as pltpu
from jax.experimental.pallas import tpu_sc as plsc

N_GRID = 50000
N_HID = 10000
C = 128
VARS = 64
TIME = 2
ATTR = 4
EDIM = 3
E = 160000

NC = 2
NS = 16
NW = NC * NS
CHUNK = 128
EPW = 5120
NCHUNK = EPW // CHUNK
EPAD = EPW * NW
SW = 16

HID_ACC = 10048

DEC_RNG = 10000
DEC_NP = 5
DEC_ACC = 10112

HID_SPLANE = 10240
DECS_RNG = 10240
DECS_NP = 5
DECS_PLANE = 10368

_f32 = jnp.float32
_i32 = jnp.int32


def _prep_edges(edge_index, edge_attr, trash_dst):
    src = edge_index[0].astype(_i32)
    dst = edge_index[1].astype(_i32)
    pad = EPAD - E
    src_p = jnp.concatenate([src, jnp.zeros((pad,), _i32)]).reshape(NW, NCHUNK, CHUNK)
    dst_p = jnp.concatenate([dst, jnp.full((pad,), trash_dst, _i32)]).reshape(NW, NCHUNK, CHUNK)
    a = jnp.concatenate([edge_attr.astype(_f32), jnp.zeros((pad, EDIM), _f32)], axis=0)
    attr_p = a.T.reshape(EDIM, NW, EPW).transpose(1, 0, 2)
    return src_p, dst_p, attr_p


def _mesh():
    return plsc.VectorSubcoreMesh(
        core_axis_name="c", subcore_axis_name="s", num_cores=NC, num_subcores=NS)


def _zero_rows(zbuf, acc, s, n_rows, zrows, zsem):
    n_chunks = n_rows // zrows

    @pl.loop(0, n_chunks // NS)
    def _z(i):
        pltpu.async_copy(zbuf, acc.at[pl.ds((i * NS + s) * zrows, zrows)], zsem)

    @pl.loop(0, n_chunks // NS)
    def _zw(i):
        pltpu.make_async_copy(zbuf, acc.at[pl.ds((i * NS + s) * zrows, zrows)],
                              zsem).wait()


def _drain_1016(acc, out_ref, s, base):
    @pl.when(s < NS - 1)
    def _d0():
        pltpu.sync_copy(acc.at[pl.ds(s * 632, 632)],
                        out_ref.at[pl.ds(base + s * 632, 632)])

    @pl.when(s == NS - 1)
    def _d1():
        pltpu.sync_copy(acc.at[pl.ds((NS - 1) * 632, 520)],
                        out_ref.at[pl.ds(base + (NS - 1) * 632, 520)])


def _sc_mp_hidden(src_p, dst_p, h):
    @functools.partial(
        pl.kernel,
        out_type=jax.ShapeDtypeStruct((NC, N_HID, C), _f32),
        mesh=_mesh(),
        compiler_params=pltpu.CompilerParams(needs_layout_passes=False),
        scratch_types=[
            pltpu.VMEM_SHARED((HID_ACC, C), _f32),
            pltpu.VMEM((NCHUNK, CHUNK), _i32),
            pltpu.VMEM((NCHUNK, CHUNK), _i32),
            pltpu.VMEM((2, CHUNK, C), _f32),
            pltpu.VMEM((8, C), _f32),
            pltpu.SemaphoreType.DMA,
            pltpu.SemaphoreType.DMA,
            pltpu.SemaphoreType.DMA,
        ],
    )
    def sc_fn(src_r, dst_r, h_r, a_out, acc, srcv, dstv, rows, zbuf,
              g0, g1, zsem):
        c = lax.axis_index("c")
        s = lax.axis_index("s")
        wid = c * NS + s
        pltpu.sync_copy(src_r.at[wid], srcv)
        pltpu.sync_copy(dst_r.at[wid], dstv)
        zv = jnp.zeros((16,), _f32)

        @pl.loop(0, 8)
        def _zb(i):
            for t in range(C // 16):
                zbuf[i, pl.ds(t * 16, 16)] = zv

        _zero_rows(zbuf, acc, s, HID_ACC, 8, zsem)
        plsc.subcore_barrier()

        gsems = (g0, g1)
        pltpu.async_copy(h_r.at[srcv.at[0]], rows.at[0], g0)
        pltpu.async_copy(h_r.at[srcv.at[1]], rows.at[1], g1)

        @pl.loop(0, NCHUNK // 2)
        def _main(i):
            for b in range(2):
                j = i * 2 + b
                pltpu.make_async_copy(h_r.at[srcv.at[j]], rows.at[b],
                                      gsems[b]).wait()
                pltpu.sync_copy(rows.at[b], acc.at[dstv.at[j]], add=True)

                @pl.when(i * 2 + b + 2 < NCHUNK)
                def _nx(j=j, b=b):
                    pltpu.async_copy(h_r.at[srcv.at[j + 2]], rows.at[b],
                                     gsems[b])

        plsc.subcore_barrier()
        _drain_1016(acc, a_out.at[c], s, 0)

    return sc_fn(src_p, dst_p, h)


def _attr_accumulate(dstv, attrv, sacc, plane, lo, rng, trash):
    @pl.loop(0, NCHUNK)
    def _ch(j):
        for t in range(CHUNK // 16):
            d = dstv[j, pl.ds(t * 16, 16)]
            if rng is None:
                base = d
            else:
                m = (d >= lo) & (d < lo + rng)
                base = jnp.where(m, d - lo, trash)
            for k in range(EDIM):
                v = attrv[k, pl.ds(j * CHUNK + t * 16, 16)]
                plsc.addupdate_scatter(sacc, [base + k * plane], v)


def _zero_flat(sacc, n):
    zv = jnp.zeros((16,), _f32)

    @pl.loop(0, n // 16)
    def _zf(i):
        sacc[pl.ds(i * 16, 16)] = zv


def _sc_attrs_hidden(dstE, attrE, dstP, attrP):
    @functools.partial(
        pl.kernel,
        out_type=(jax.ShapeDtypeStruct((NW, (EDIM + 1) * HID_SPLANE), _f32),
                  jax.ShapeDtypeStruct((NW, (EDIM + 1) * HID_SPLANE), _f32)),
        mesh=_mesh(),
        compiler_params=pltpu.CompilerParams(needs_layout_passes=False),
        scratch_types=[
            pltpu.VMEM(((EDIM + 1) * HID_SPLANE,), _f32),
            pltpu.VMEM(((EDIM + 1) * HID_SPLANE,), _f32),
            pltpu.VMEM((NCHUNK, CHUNK), _i32),
            pltpu.VMEM((EDIM, EPW), _f32),
        ],
    )
    def sc_fn(dstE_r, attrE_r, dstP_r, attrP_r, sE_out, sP_out,
              saccE, saccP, dstv, attrv):
        c = lax.axis_index("c")
        s = lax.axis_index("s")
        wid = c * NS + s
        for dst_r, attr_r, sacc, out in ((dstE_r, attrE_r, saccE, sE_out),
                                         (dstP_r, attrP_r, saccP, sP_out)):
            pltpu.sync_copy(dst_r.at[wid], dstv)
            pltpu.sync_copy(attr_r.at[wid], attrv)
            _zero_flat(sacc, (EDIM + 1) * HID_SPLANE)
            _attr_accumulate(dstv, attrv, sacc, HID_SPLANE, 0, None, None)
            pltpu.sync_copy(sacc, out.at[wid])

    return sc_fn(dstE, attrE, dstP, attrP)


def _sc_attrs_dec(dstD, attrD):
    @functools.partial(
        pl.kernel,
        out_type=jax.ShapeDtypeStruct((NW, DECS_NP * (EDIM + 1) * DECS_PLANE), _f32),
        mesh=_mesh(),
        compiler_params=pltpu.CompilerParams(needs_layout_passes=False),
        scratch_types=[
            pltpu.VMEM(((EDIM + 1) * DECS_PLANE,), _f32),
            pltpu.VMEM((NCHUNK, CHUNK), _i32),
            pltpu.VMEM((EDIM, EPW), _f32),
        ],
    )
    def sc_fn(dst_r, attr_r, s_out, sacc, dstv, attrv):
        c = lax.axis_index("c")
        s = lax.axis_index("s")
        wid = c * NS + s
        pltpu.sync_copy(dst_r.at[wid], dstv)
        pltpu.sync_copy(attr_r.at[wid], attrv)
        trash = jnp.full((16,), DECS_RNG, _i32)
        for r in range(DECS_NP):
            _zero_flat(sacc, (EDIM + 1) * DECS_PLANE)
            _attr_accumulate(dstv, attrv, sacc, DECS_PLANE, r * DECS_RNG,
                             DECS_RNG, trash)
            pltpu.sync_copy(sacc, s_out.at[wid, pl.ds(r * (EDIM + 1) * DECS_PLANE, (EDIM + 1) * DECS_PLANE)])

    return sc_fn(dstD, attrD)


def _sc_mp_dec(pk_p, h):
    @functools.partial(
        pl.kernel,
        out_type=jax.ShapeDtypeStruct((NC, DEC_NP * DEC_RNG, C), _f32),
        mesh=_mesh(),
        compiler_params=pltpu.CompilerParams(needs_layout_passes=False),
        scratch_types=[
            pltpu.VMEM_SHARED((DEC_ACC, C), _f32),
            pltpu.VMEM((NCHUNK, CHUNK), _i32),
            pltpu.VMEM((NCHUNK, CHUNK), _i32),
            pltpu.VMEM((NCHUNK, CHUNK), _i32),
            pltpu.VMEM((2, CHUNK, C), _f32),
            pltpu.VMEM((8, C), _f32),
            pltpu.SemaphoreType.DMA,
            pltpu.SemaphoreType.DMA,
            pltpu.SemaphoreType.DMA,
        ],
    )
    def sc_fn(pk_r, h_r, a_out,
              acc, pkv, scomp, dcomp, rows, zbuf, g0, g1, zsem):
        c = lax.axis_index("c")
        s = lax.axis_index("s")
        wid = c * NS + s
        pltpu.sync_copy(pk_r.at[wid], pkv)
        zv = jnp.zeros((16,), _f32)
        z16i = jnp.zeros((16,), _i32)
        trash16 = jnp.full((16,), DEC_RNG, _i32)

        @pl.loop(0, 8)
        def _zb(i):
            for t in range(C // 16):
                zbuf[i, pl.ds(t * 16, 16)] = zv

        for r in range(DEC_NP):
            lo = r * DEC_RNG
            _zero_rows(zbuf, acc, s, DEC_ACC, 8, zsem)

            @pl.loop(0, NCHUNK)
            def _pf(j):
                for t in range(CHUNK // 16):
                    dcomp[j, pl.ds(t * 16, 16)] = trash16
                    scomp[j, pl.ds(t * 16, 16)] = z16i

            plsc.subcore_barrier()

            def cbody(j, cur):
                for t in range(CHUNK // 16):
                    w = pkv[j, pl.ds(t * 16, 16)]
                    d = lax.shift_right_logical(w, 16)
                    m = (d >= lo) & (d < lo + DEC_RNG)
                    dl = d - lo
                    pos = jnp.maximum(cur + plsc.cumsum(m.astype(_i32)) - 1, 0)
                    row = lax.shift_right_logical(pos, 7)
                    col = lax.bitwise_and(pos, 127)
                    sv = lax.bitwise_and(w, 0xFFFF)
                    plsc.store_scatter(dcomp, [row, col], dl, mask=m)
                    plsc.store_scatter(scomp, [row, col], sv, mask=m)
                    cur = cur + plsc.all_reduce_population_count(m)
                return cur

            cur = lax.fori_loop(0, NCHUNK, cbody, jnp.zeros((16,), _i32))
            nch = lax.shift_right_logical(jnp.max(cur) + (CHUNK - 1), 7)
            npair = jnp.maximum(lax.shift_right_logical(nch + 1, 1), 1)
            nch_pad = npair * 2
            gsems = (g0, g1)
            pltpu.async_copy(h_r.at[scomp.at[0]], rows.at[0], g0)
            pltpu.async_copy(h_r.at[scomp.at[1]], rows.at[1], g1)

            def gbody(i, carry):
                for b in range(2):
                    j = i * 2 + b
                    pltpu.make_async_copy(h_r.at[scomp.at[j]], rows.at[b],
                                          gsems[b]).wait()
                    pltpu.sync_copy(rows.at[b], acc.at[dcomp.at[j]], add=True)

                    @pl.when(j + 2 < nch_pad)
                    def _nx(j=j, b=b):
                        pltpu.async_copy(h_r.at[scomp.at[j + 2]], rows.at[b],
                                         gsems[b])

                return carry

            lax.fori_loop(0, npair, gbody, 0)
            plsc.subcore_barrier()
            _drain_1016(acc, a_out.at[c], s, lo)
            plsc.subcore_barrier()

    return sc_fn(pk_p, h)


def _dot(a, b):
    return jnp.dot(a, b, preferred_element_type=_f32)


def _embed_call(x0, x1, ga, w_src, wm):
    BLK = 1000

    def body(x0_r, x1_r, ga_r, w_r, wm_r, xd_r, he_r):
        xd = (_dot(x0_r[...], w_r[0:VARS])
              + _dot(x1_r[...], w_r[VARS:2 * VARS])
              + _dot(ga_r[...], w_r[2 * VARS:2 * VARS + ATTR]))
        xd_r[...] = xd
        he_r[...] = _dot(xd, wm_r[...])

    return pl.pallas_call(
        body,
        grid=(N_GRID // BLK,),
        in_specs=[
            pl.BlockSpec((BLK, VARS), lambda i: (i, 0)),
            pl.BlockSpec((BLK, VARS), lambda i: (i, 0)),
            pl.BlockSpec((BLK, ATTR), lambda i: (i, 0)),
            pl.BlockSpec((TIME * VARS + ATTR, C), lambda i: (0, 0)),
            pl.BlockSpec((C, C), lambda i: (0, 0)),
        ],
        out_specs=(pl.BlockSpec((BLK, C), lambda i: (i, 0)),
                   pl.BlockSpec((BLK, C), lambda i: (i, 0))),
        out_shape=(jax.ShapeDtypeStruct((N_GRID, C), _f32),
                   jax.ShapeDtypeStruct((N_GRID, C), _f32)),
    )(x0, x1, ga, w_src, wm)


def _s_term(t, s_blk, wewu):
    ssum = jnp.sum(s_blk, axis=0)
    for k in range(EDIM):
        t = t + ssum[k][:, None] * wewu[k][None, :]
    return t


def _post_enc_call(A, S, ha, we, wu, wdst, ws, wm0):
    BLK = 2048

    def body(a_r, s_r, ha_r, we_r, wu_r, wd_r, ws_r, wm_r, xh_r, h_r):
        t = _dot(a_r[0] + a_r[1], wu_r[...])
        t = _s_term(t, s_r[...], _dot(we_r[...], wu_r[...]))
        t = t + _dot(ha_r[...], _dot(wd_r[...], ws_r[...]))
        xh = jnp.maximum(t, 0.0)
        xh_r[...] = xh
        h_r[...] = _dot(xh, wm_r[...])

    return pl.pallas_call(
        body,
        grid=(pl.cdiv(N_HID, BLK),),
        in_specs=[
            pl.BlockSpec((NC, BLK, C), lambda i: (0, i, 0)),
            pl.BlockSpec((NW, EDIM + 1, BLK), lambda i: (0, 0, i)),
            pl.BlockSpec((BLK, ATTR), lambda i: (i, 0)),
            pl.BlockSpec((EDIM, C), lambda i: (0, 0)),
            pl.BlockSpec((C, C), lambda i: (0, 0)),
            pl.BlockSpec((ATTR, C), lambda i: (0, 0)),
            pl.BlockSpec((C, C), lambda i: (0, 0)),
            pl.BlockSpec((C, C), lambda i: (0, 0)),
        ],
        out_specs=(pl.BlockSpec((BLK, C), lambda i: (i, 0)),
                   pl.BlockSpec((BLK, C), lambda i: (i, 0))),
        out_shape=(jax.ShapeDtypeStruct((N_HID, C), _f32),
                   jax.ShapeDtypeStruct((N_HID, C), _f32)),
    )(A, S, ha, we, wu, wdst, ws, wm0)


def _post_proc_call(A, S, xh, we, wu, ws, wm_next):
    BLK = 2048

    def body(a_r, s_r, xh_r, we_r, wu_r, ws_r, wm_r, xo_r, h_r):
        xh = xh_r[...]
        t = _dot(a_r[0] + a_r[1], wu_r[...])
        t = _s_term(t, s_r[...], _dot(we_r[...], wu_r[...]))
        t = t + _dot(xh, ws_r[...])
        xo = xh + jnp.maximum(t, 0.0)
        xo_r[...] = xo
        h_r[...] = _dot(xo, wm_r[...])

    return pl.pallas_call(
        body,
        grid=(pl.cdiv(N_HID, BLK),),
        in_specs=[
            pl.BlockSpec((NC, BLK, C), lambda i: (0, i, 0)),
            pl.BlockSpec((NW, EDIM + 1, BLK), lambda i: (0, 0, i)),
            pl.BlockSpec((BLK, C), lambda i: (i, 0)),
            pl.BlockSpec((EDIM, C), lambda i: (0, 0)),
            pl.BlockSpec((C, C), lambda i: (0, 0)),
            pl.BlockSpec((C, C), lambda i: (0, 0)),
            pl.BlockSpec((C, C), lambda i: (0, 0)),
        ],
        out_specs=(pl.BlockSpec((BLK, C), lambda i: (i, 0)),
                   pl.BlockSpec((BLK, C), lambda i: (i, 0))),
        out_shape=(jax.ShapeDtypeStruct((N_HID, C), _f32),
                   jax.ShapeDtypeStruct((N_HID, C), _f32)),
    )(A, S, xh, we, wu, ws, wm_next)


def _final_call(A, S, xd, we, wu, ws, wout):
    BLK = 2048

    def body(a_r, s_r, xd_r, we_r, wu_r, ws_r, wo_r, o_r):
        t = _dot(a_r[0] + a_r[1], wu_r[...])
        t = _s_term(t, s_r[:, 0], _dot(we_r[...], wu_r[...]))
        t = t + _dot(xd_r[...], ws_r[...])
        o_r[...] = _dot(jnp.maximum(t, 0.0), wo_r[...])

    return pl.pallas_call(
        body,
        grid=(pl.cdiv(N_GRID, BLK),),
        in_specs=[
            pl.BlockSpec((NC, BLK, C), lambda i: (0, i, 0)),
            pl.BlockSpec((NW, 1, EDIM + 1, BLK), lambda i: (0, i // 5, 0, i % 5)),
            pl.BlockSpec((BLK, C), lambda i: (i, 0)),
            pl.BlockSpec((EDIM, C), lambda i: (0, 0)),
            pl.BlockSpec((C, C), lambda i: (0, 0)),
            pl.BlockSpec((C, C), lambda i: (0, 0)),
            pl.BlockSpec((C, VARS), lambda i: (0, 0)),
        ],
        out_specs=pl.BlockSpec((BLK, VARS), lambda i: (i, 0)),
        out_shape=jax.ShapeDtypeStruct((N_GRID, VARS), _f32),
    )(A, S, xd, we, wu, ws, wout)


def kernel(x, edge_index_enc, edge_attr_enc, edge_index_proc, edge_attr_proc,
           edge_index_dec, edge_attr_dec, grid_attrs, hidden_attrs,
           W_src_emb, W_dst_emb,
           enc_Wm, enc_We, enc_Wu, enc_Ws,
           proc_Wm, proc_We, proc_Wu, proc_Ws,
           dec_Wm, dec_We, dec_Wu, dec_Ws, W_out):
    x0 = x[0, 0, 0]
    x1 = x[0, 1, 0]
    srcE, dstE, attrE = _prep_edges(edge_index_enc, edge_attr_enc, N_HID)
    srcP, dstP, attrP = _prep_edges(edge_index_proc, edge_attr_proc, N_HID)
    srcD, dstD, attrD = _prep_edges(edge_index_dec, edge_attr_dec, N_GRID)
    pk = jnp.bitwise_or(
        jnp.left_shift(edge_index_dec[1].astype(_i32), 16),
        edge_index_dec[0].astype(_i32))
    pkD = jnp.concatenate(
        [pk, jnp.full((EPAD - E,), N_GRID << 16, _i32)]).reshape(NW, NCHUNK, CHUNK)

    x_data, h = _embed_call(x0, x1, grid_attrs, W_src_emb, enc_Wm)

    Se, Sp = _sc_attrs_hidden(dstE, attrE, dstP, attrP)
    Se = Se.reshape(NW, EDIM + 1, HID_SPLANE)
    Sp = Sp.reshape(NW, EDIM + 1, HID_SPLANE)
    Sd = _sc_attrs_dec(dstD, attrD).reshape(NW, DECS_NP, EDIM + 1, DECS_PLANE)

    A = _sc_mp_hidden(srcE, dstE, h)
    x_hid, h = _post_enc_call(A, Se, hidden_attrs, enc_We, enc_Wu,
                              W_dst_emb, enc_Ws, proc_Wm[0])

    for l in range(2):
        A = _sc_mp_hidden(srcP, dstP, h)
        wm_next = proc_Wm[1] if l == 0 else dec_Wm
        x_hid, h = _post_proc_call(A, Sp, x_hid, proc_We[l], proc_Wu[l],
                                   proc_Ws[l], wm_next)

    A = _sc_mp_dec(pkD, h)
    out = _final_call(A, Sd, x_data, dec_We, dec_Wu, dec_Ws, W_out)
    return out.reshape(1, 1, N_GRID, VARS)


# speedup vs baseline: 1.1918x; 1.1918x over previous
import functools

import jax
import jax.numpy as jnp
from jax import lax
from jax.experimental import pallas as pl
from jax.experimental.pallas import tpu as pltpu
from jax.experimental.pallas import tpu_sc as plsc

N_GRID = 50000
N_HID = 10000
C = 128
VARS = 64
TIME = 2
ATTR = 4
EDIM = 3
E = 160000

NC = 2
NS = 16
NW = NC * NS
CHUNK = 128
EPW = 5120
NCHUNK = EPW // CHUNK
EPAD = EPW * NW
SW = 16

HID_ACC = 10048

DEC_RNG = 10000
DEC_NP = 5
DEC_ACC = 10112

HID_SPLANE = 10240
DECS_RNG = 10240
DECS_NP = 5
DECS_PLANE = 10368

_f32 = jnp.float32
_i32 = jnp.int32


def _prep_edges(edge_index, edge_attr, trash_dst):
    src = edge_index[0].astype(_i32)
    dst = edge_index[1].astype(_i32)
    pad = EPAD - E
    src_p = jnp.concatenate([src, jnp.zeros((pad,), _i32)]).reshape(NW, NCHUNK, CHUNK)
    dst_p = jnp.concatenate([dst, jnp.full((pad,), trash_dst, _i32)]).reshape(NW, NCHUNK, CHUNK)
    a = jnp.concatenate([edge_attr.astype(_f32), jnp.zeros((pad, EDIM), _f32)], axis=0)
    attr_p = a.T.reshape(EDIM, NW, EPW).transpose(1, 0, 2)
    return src_p, dst_p, attr_p


def _mesh():
    return plsc.VectorSubcoreMesh(
        core_axis_name="c", subcore_axis_name="s", num_cores=NC, num_subcores=NS)


def _zero_rows(zbuf, acc, s, n_rows, zrows, zsem):
    n_chunks = n_rows // zrows

    @pl.loop(0, n_chunks // NS)
    def _z(i):
        pltpu.async_copy(zbuf, acc.at[pl.ds((i * NS + s) * zrows, zrows)], zsem)

    @pl.loop(0, n_chunks // NS)
    def _zw(i):
        pltpu.make_async_copy(zbuf, acc.at[pl.ds((i * NS + s) * zrows, zrows)],
                              zsem).wait()


def _drain_1016(acc, out_ref, s, base):
    @pl.when(s < NS - 1)
    def _d0():
        pltpu.sync_copy(acc.at[pl.ds(s * 632, 632)],
                        out_ref.at[pl.ds(base + s * 632, 632)])

    @pl.when(s == NS - 1)
    def _d1():
        pltpu.sync_copy(acc.at[pl.ds((NS - 1) * 632, 520)],
                        out_ref.at[pl.ds(base + (NS - 1) * 632, 520)])


def _sc_mp_hidden(src_p, dst_p, h):
    @functools.partial(
        pl.kernel,
        out_type=jax.ShapeDtypeStruct((NC, N_HID, C), _f32),
        mesh=_mesh(),
        compiler_params=pltpu.CompilerParams(needs_layout_passes=False),
        scratch_types=[
            pltpu.VMEM_SHARED((HID_ACC, C), _f32),
            pltpu.VMEM((NCHUNK, CHUNK), _i32),
            pltpu.VMEM((NCHUNK, CHUNK), _i32),
            pltpu.VMEM((2, CHUNK, C), _f32),
            pltpu.VMEM((8, C), _f32),
            pltpu.SemaphoreType.DMA,
            pltpu.SemaphoreType.DMA,
            pltpu.SemaphoreType.DMA,
        ],
    )
    def sc_fn(src_r, dst_r, h_r, a_out, acc, srcv, dstv, rows, zbuf,
              g0, g1, zsem):
        c = lax.axis_index("c")
        s = lax.axis_index("s")
        wid = c * NS + s
        pltpu.sync_copy(src_r.at[wid], srcv)
        pltpu.sync_copy(dst_r.at[wid], dstv)
        zv = jnp.zeros((16,), _f32)

        @pl.loop(0, 8)
        def _zb(i):
            for t in range(C // 16):
                zbuf[i, pl.ds(t * 16, 16)] = zv

        _zero_rows(zbuf, acc, s, HID_ACC, 8, zsem)
        plsc.subcore_barrier()

        gsems = (g0, g1)
        pltpu.async_copy(h_r.at[srcv.at[0]], rows.at[0], g0)
        pltpu.async_copy(h_r.at[srcv.at[1]], rows.at[1], g1)

        @pl.loop(0, NCHUNK // 2)
        def _main(i):
            for b in range(2):
                j = i * 2 + b
                pltpu.make_async_copy(h_r.at[srcv.at[j]], rows.at[b],
                                      gsems[b]).wait()
                pltpu.sync_copy(rows.at[b], acc.at[dstv.at[j]], add=True)

                @pl.when(i * 2 + b + 2 < NCHUNK)
                def _nx(j=j, b=b):
                    pltpu.async_copy(h_r.at[srcv.at[j + 2]], rows.at[b],
                                     gsems[b])

        plsc.subcore_barrier()
        _drain_1016(acc, a_out.at[c], s, 0)

    return sc_fn(src_p, dst_p, h)


def _attr_accumulate(dstv, attrv, sacc, plane, lo, rng, trash):
    @pl.loop(0, NCHUNK)
    def _ch(j):
        for t in range(CHUNK // 16):
            d = dstv[j, pl.ds(t * 16, 16)]
            if rng is None:
                base = d
            else:
                m = (d >= lo) & (d < lo + rng)
                base = jnp.where(m, d - lo, trash)
            for k in range(EDIM):
                v = attrv[k, pl.ds(j * CHUNK + t * 16, 16)]
                plsc.addupdate_scatter(sacc, [base + k * plane], v)


def _zero_flat(sacc, n):
    zv = jnp.zeros((16,), _f32)

    @pl.loop(0, n // 16)
    def _zf(i):
        sacc[pl.ds(i * 16, 16)] = zv


def _sc_attrs_hidden(dstE, attrE, dstP, attrP):
    @functools.partial(
        pl.kernel,
        out_type=(jax.ShapeDtypeStruct((NW, (EDIM + 1) * HID_SPLANE), _f32),
                  jax.ShapeDtypeStruct((NW, (EDIM + 1) * HID_SPLANE), _f32)),
        mesh=_mesh(),
        compiler_params=pltpu.CompilerParams(needs_layout_passes=False),
        scratch_types=[
            pltpu.VMEM(((EDIM + 1) * HID_SPLANE,), _f32),
            pltpu.VMEM(((EDIM + 1) * HID_SPLANE,), _f32),
            pltpu.VMEM((NCHUNK, CHUNK), _i32),
            pltpu.VMEM((EDIM, EPW), _f32),
        ],
    )
    def sc_fn(dstE_r, attrE_r, dstP_r, attrP_r, sE_out, sP_out,
              saccE, saccP, dstv, attrv):
        c = lax.axis_index("c")
        s = lax.axis_index("s")
        wid = c * NS + s
        for dst_r, attr_r, sacc, out in ((dstE_r, attrE_r, saccE, sE_out),
                                         (dstP_r, attrP_r, saccP, sP_out)):
            pltpu.sync_copy(dst_r.at[wid], dstv)
            pltpu.sync_copy(attr_r.at[wid], attrv)
            _zero_flat(sacc, (EDIM + 1) * HID_SPLANE)
            _attr_accumulate(dstv, attrv, sacc, HID_SPLANE, 0, None, None)
            pltpu.sync_copy(sacc, out.at[wid])

    return sc_fn(dstE, attrE, dstP, attrP)


def _sc_attrs_dec(dstD, attrD):
    @functools.partial(
        pl.kernel,
        out_type=jax.ShapeDtypeStruct((NW, DECS_NP * (EDIM + 1) * DECS_PLANE), _f32),
        mesh=_mesh(),
        compiler_params=pltpu.CompilerParams(needs_layout_passes=False),
        scratch_types=[
            pltpu.VMEM(((EDIM + 1) * DECS_PLANE,), _f32),
            pltpu.VMEM((NCHUNK, CHUNK), _i32),
            pltpu.VMEM((EDIM, EPW), _f32),
        ],
    )
    def sc_fn(dst_r, attr_r, s_out, sacc, dstv, attrv):
        c = lax.axis_index("c")
        s = lax.axis_index("s")
        wid = c * NS + s
        pltpu.sync_copy(dst_r.at[wid], dstv)
        pltpu.sync_copy(attr_r.at[wid], attrv)
        trash = jnp.full((16,), DECS_RNG, _i32)
        for r in range(DECS_NP):
            _zero_flat(sacc, (EDIM + 1) * DECS_PLANE)
            _attr_accumulate(dstv, attrv, sacc, DECS_PLANE, r * DECS_RNG,
                             DECS_RNG, trash)
            pltpu.sync_copy(sacc, s_out.at[wid, pl.ds(r * (EDIM + 1) * DECS_PLANE, (EDIM + 1) * DECS_PLANE)])

    return sc_fn(dstD, attrD)


def _sc_mp_dec(pk_p, h):
    @functools.partial(
        pl.kernel,
        out_type=jax.ShapeDtypeStruct((NC, DEC_NP * DEC_RNG, C), _f32),
        mesh=_mesh(),
        compiler_params=pltpu.CompilerParams(needs_layout_passes=False),
        scratch_types=[
            pltpu.VMEM_SHARED((DEC_ACC, C), _f32),
            pltpu.VMEM((NCHUNK, CHUNK), _i32),
            pltpu.VMEM((NCHUNK, CHUNK), _i32),
            pltpu.VMEM((NCHUNK, CHUNK), _i32),
            pltpu.VMEM((2, CHUNK, C), _f32),
            pltpu.VMEM((8, C), _f32),
            pltpu.SemaphoreType.DMA,
            pltpu.SemaphoreType.DMA,
            pltpu.SemaphoreType.DMA,
        ],
    )
    def sc_fn(pk_r, h_r, a_out,
              acc, pkv, scomp, dcomp, rows, zbuf, g0, g1, zsem):
        c = lax.axis_index("c")
        s = lax.axis_index("s")
        wid = c * NS + s
        pltpu.sync_copy(pk_r.at[wid], pkv)
        zv = jnp.zeros((16,), _f32)
        z16i = jnp.zeros((16,), _i32)
        trash16 = jnp.full((16,), DEC_RNG, _i32)

        @pl.loop(0, 8)
        def _zb(i):
            for t in range(C // 16):
                zbuf[i, pl.ds(t * 16, 16)] = zv

        for r in range(DEC_NP):
            lo = r * DEC_RNG
            _zero_rows(zbuf, acc, s, DEC_ACC, 8, zsem)

            @pl.loop(0, NCHUNK)
            def _pf(j):
                for t in range(CHUNK // 16):
                    dcomp[j, pl.ds(t * 16, 16)] = trash16
                    scomp[j, pl.ds(t * 16, 16)] = z16i

            plsc.subcore_barrier()

            def cbody(j, cur):
                for t in range(CHUNK // 16):
                    w = pkv[j, pl.ds(t * 16, 16)]
                    d = lax.shift_right_logical(w, 16)
                    m = (d >= lo) & (d < lo + DEC_RNG)
                    dl = d - lo
                    pos = jnp.maximum(cur + plsc.cumsum(m.astype(_i32)) - 1, 0)
                    row = lax.shift_right_logical(pos, 7)
                    col = lax.bitwise_and(pos, 127)
                    sv = lax.bitwise_and(w, 0xFFFF)
                    plsc.store_scatter(dcomp, [row, col], dl, mask=m)
                    plsc.store_scatter(scomp, [row, col], sv, mask=m)
                    cur = cur + plsc.all_reduce_population_count(m)
                return cur

            cur = lax.fori_loop(0, NCHUNK, cbody, jnp.zeros((16,), _i32))
            nch = lax.shift_right_logical(jnp.max(cur) + (CHUNK - 1), 7)

            def gbody(j, carry):
                cp = pltpu.async_copy(h_r.at[scomp.at[j]], rows.at[0], g0)
                cp.wait()
                pltpu.sync_copy(rows.at[0], acc.at[dcomp.at[j]], add=True)
                return carry

            lax.fori_loop(0, nch, gbody, 0)
            plsc.subcore_barrier()
            _drain_1016(acc, a_out.at[c], s, lo)
            plsc.subcore_barrier()

    return sc_fn(pk_p, h)


def _dot(a, b):
    return jnp.dot(a, b, preferred_element_type=_f32)


def _embed_call(x0, x1, ga, w_src, wm):
    BLK = 1000

    def body(x0_r, x1_r, ga_r, w_r, wm_r, xd_r, he_r):
        xd = (_dot(x0_r[...], w_r[0:VARS])
              + _dot(x1_r[...], w_r[VARS:2 * VARS])
              + _dot(ga_r[...], w_r[2 * VARS:2 * VARS + ATTR]))
        xd_r[...] = xd
        he_r[...] = _dot(xd, wm_r[...])

    return pl.pallas_call(
        body,
        grid=(N_GRID // BLK,),
        in_specs=[
            pl.BlockSpec((BLK, VARS), lambda i: (i, 0)),
            pl.BlockSpec((BLK, VARS), lambda i: (i, 0)),
            pl.BlockSpec((BLK, ATTR), lambda i: (i, 0)),
            pl.BlockSpec((TIME * VARS + ATTR, C), lambda i: (0, 0)),
            pl.BlockSpec((C, C), lambda i: (0, 0)),
        ],
        out_specs=(pl.BlockSpec((BLK, C), lambda i: (i, 0)),
                   pl.BlockSpec((BLK, C), lambda i: (i, 0))),
        out_shape=(jax.ShapeDtypeStruct((N_GRID, C), _f32),
                   jax.ShapeDtypeStruct((N_GRID, C), _f32)),
    )(x0, x1, ga, w_src, wm)


def _s_term(t, s_blk, wewu):
    ssum = jnp.sum(s_blk, axis=0)
    for k in range(EDIM):
        t = t + ssum[k][:, None] * wewu[k][None, :]
    return t


def _post_enc_call(A, S, ha, we, wu, wdst, ws, wm0):
    BLK = 2048

    def body(a_r, s_r, ha_r, we_r, wu_r, wd_r, ws_r, wm_r, xh_r, h_r):
        t = _dot(a_r[0] + a_r[1], wu_r[...])
        t = _s_term(t, s_r[...], _dot(we_r[...], wu_r[...]))
        t = t + _dot(ha_r[...], _dot(wd_r[...], ws_r[...]))
        xh = jnp.maximum(t, 0.0)
        xh_r[...] = xh
        h_r[...] = _dot(xh, wm_r[...])

    return pl.pallas_call(
        body,
        grid=(pl.cdiv(N_HID, BLK),),
        in_specs=[
            pl.BlockSpec((NC, BLK, C), lambda i: (0, i, 0)),
            pl.BlockSpec((NW, EDIM + 1, BLK), lambda i: (0, 0, i)),
            pl.BlockSpec((BLK, ATTR), lambda i: (i, 0)),
            pl.BlockSpec((EDIM, C), lambda i: (0, 0)),
            pl.BlockSpec((C, C), lambda i: (0, 0)),
            pl.BlockSpec((ATTR, C), lambda i: (0, 0)),
            pl.BlockSpec((C, C), lambda i: (0, 0)),
            pl.BlockSpec((C, C), lambda i: (0, 0)),
        ],
        out_specs=(pl.BlockSpec((BLK, C), lambda i: (i, 0)),
                   pl.BlockSpec((BLK, C), lambda i: (i, 0))),
        out_shape=(jax.ShapeDtypeStruct((N_HID, C), _f32),
                   jax.ShapeDtypeStruct((N_HID, C), _f32)),
    )(A, S, ha, we, wu, wdst, ws, wm0)


def _post_proc_call(A, S, xh, we, wu, ws, wm_next):
    BLK = 2048

    def body(a_r, s_r, xh_r, we_r, wu_r, ws_r, wm_r, xo_r, h_r):
        xh = xh_r[...]
        t = _dot(a_r[0] + a_r[1], wu_r[...])
        t = _s_term(t, s_r[...], _dot(we_r[...], wu_r[...]))
        t = t + _dot(xh, ws_r[...])
        xo = xh + jnp.maximum(t, 0.0)
        xo_r[...] = xo
        h_r[...] = _dot(xo, wm_r[...])

    return pl.pallas_call(
        body,
        grid=(pl.cdiv(N_HID, BLK),),
        in_specs=[
            pl.BlockSpec((NC, BLK, C), lambda i: (0, i, 0)),
            pl.BlockSpec((NW, EDIM + 1, BLK), lambda i: (0, 0, i)),
            pl.BlockSpec((BLK, C), lambda i: (i, 0)),
            pl.BlockSpec((EDIM, C), lambda i: (0, 0)),
            pl.BlockSpec((C, C), lambda i: (0, 0)),
            pl.BlockSpec((C, C), lambda i: (0, 0)),
            pl.BlockSpec((C, C), lambda i: (0, 0)),
        ],
        out_specs=(pl.BlockSpec((BLK, C), lambda i: (i, 0)),
                   pl.BlockSpec((BLK, C), lambda i: (i, 0))),
        out_shape=(jax.ShapeDtypeStruct((N_HID, C), _f32),
                   jax.ShapeDtypeStruct((N_HID, C), _f32)),
    )(A, S, xh, we, wu, ws, wm_next)


def _final_call(A, S, xd, we, wu, ws, wout):
    BLK = 2048

    def body(a_r, s_r, xd_r, we_r, wu_r, ws_r, wo_r, o_r):
        t = _dot(a_r[0] + a_r[1], wu_r[...])
        t = _s_term(t, s_r[:, 0], _dot(we_r[...], wu_r[...]))
        t = t + _dot(xd_r[...], ws_r[...])
        o_r[...] = _dot(jnp.maximum(t, 0.0), wo_r[...])

    return pl.pallas_call(
        body,
        grid=(pl.cdiv(N_GRID, BLK),),
        in_specs=[
            pl.BlockSpec((NC, BLK, C), lambda i: (0, i, 0)),
            pl.BlockSpec((NW, 1, EDIM + 1, BLK), lambda i: (0, i // 5, 0, i % 5)),
            pl.BlockSpec((BLK, C), lambda i: (i, 0)),
            pl.BlockSpec((EDIM, C), lambda i: (0, 0)),
            pl.BlockSpec((C, C), lambda i: (0, 0)),
            pl.BlockSpec((C, C), lambda i: (0, 0)),
            pl.BlockSpec((C, VARS), lambda i: (0, 0)),
        ],
        out_specs=pl.BlockSpec((BLK, VARS), lambda i: (i, 0)),
        out_shape=jax.ShapeDtypeStruct((N_GRID, VARS), _f32),
    )(A, S, xd, we, wu, ws, wout)


def kernel(x, edge_index_enc, edge_attr_enc, edge_index_proc, edge_attr_proc,
           edge_index_dec, edge_attr_dec, grid_attrs, hidden_attrs,
           W_src_emb, W_dst_emb,
           enc_Wm, enc_We, enc_Wu, enc_Ws,
           proc_Wm, proc_We, proc_Wu, proc_Ws,
           dec_Wm, dec_We, dec_Wu, dec_Ws, W_out):
    x0 = x[0, 0, 0]
    x1 = x[0, 1, 0]
    srcE, dstE, attrE = _prep_edges(edge_index_enc, edge_attr_enc, N_HID)
    srcP, dstP, attrP = _prep_edges(edge_index_proc, edge_attr_proc, N_HID)
    srcD, dstD, attrD = _prep_edges(edge_index_dec, edge_attr_dec, N_GRID)
    pk = jnp.bitwise_or(
        jnp.left_shift(edge_index_dec[1].astype(_i32), 16),
        edge_index_dec[0].astype(_i32))
    pkD = jnp.concatenate(
        [pk, jnp.full((EPAD - E,), N_GRID << 16, _i32)]).reshape(NW, NCHUNK, CHUNK)

    x_data, h = _embed_call(x0, x1, grid_attrs, W_src_emb, enc_Wm)

    Se, Sp = _sc_attrs_hidden(dstE, attrE, dstP, attrP)
    Se = Se.reshape(NW, EDIM + 1, HID_SPLANE)
    Sp = Sp.reshape(NW, EDIM + 1, HID_SPLANE)
    Sd = _sc_attrs_dec(dstD, attrD).reshape(NW, DECS_NP, EDIM + 1, DECS_PLANE)

    A = _sc_mp_hidden(srcE, dstE, h)
    x_hid, h = _post_enc_call(A, Se, hidden_attrs, enc_We, enc_Wu,
                              W_dst_emb, enc_Ws, proc_Wm[0])

    for l in range(2):
        A = _sc_mp_hidden(srcP, dstP, h)
        wm_next = proc_Wm[1] if l == 0 else dec_Wm
        x_hid, h = _post_proc_call(A, Sp, x_hid, proc_We[l], proc_Wu[l],
                                   proc_Ws[l], wm_next)

    A = _sc_mp_dec(pkD, h)
    out = _final_call(A, Sd, x_data, dec_We, dec_Wu, dec_Ws, W_out)
    return out.reshape(1, 1, N_GRID, VARS)


# speedup vs baseline: 1.1923x; 1.0004x over previous
import functools

import jax
import jax.numpy as jnp
from jax import lax
from jax.experimental import pallas as pl
from jax.experimental.pallas import tpu as pltpu
from jax.experimental.pallas import tpu_sc as plsc

N_GRID = 50000
N_HID = 10000
C = 128
VARS = 64
TIME = 2
ATTR = 4
EDIM = 3
E = 160000

NC = 2
NS = 16
NW = NC * NS
CHUNK = 128
EPW = 5120
NCHUNK = EPW // CHUNK
EPAD = EPW * NW
SW = 16

HID_ACC = 10240

DEC_RNG = 10000
DEC_NP = 5
DEC_ACC = 10240

HID_SPLANE = 10240
DECS_RNG = 10240
DECS_NP = 5
DECS_PLANE = 10368

_f32 = jnp.float32
_i32 = jnp.int32


def _prep_edges(edge_index, edge_attr, trash_dst):
    src = edge_index[0].astype(_i32)
    dst = edge_index[1].astype(_i32)
    pad = EPAD - E
    src_p = jnp.concatenate([src, jnp.zeros((pad,), _i32)]).reshape(NW, NCHUNK, CHUNK)
    dst_p = jnp.concatenate([dst, jnp.full((pad,), trash_dst, _i32)]).reshape(NW, NCHUNK, CHUNK)
    a = jnp.concatenate([edge_attr.astype(_f32), jnp.zeros((pad, EDIM), _f32)], axis=0)
    attr_p = a.T.reshape(EDIM, NW, EPW).transpose(1, 0, 2)
    return src_p, dst_p, attr_p


def _mesh():
    return plsc.VectorSubcoreMesh(
        core_axis_name="c", subcore_axis_name="s", num_cores=NC, num_subcores=NS)


def _zero_rows(zbuf, acc, s, n_rows, zrows, zsem):
    n_chunks = n_rows // zrows
    assert n_chunks % NS == 0

    @pl.loop(0, n_chunks // NS)
    def _z(i):
        pltpu.async_copy(zbuf, acc.at[pl.ds((i * NS + s) * zrows, zrows)], zsem)

    @pl.loop(0, n_chunks // NS)
    def _zw(i):
        pltpu.make_async_copy(zbuf, acc.at[pl.ds((i * NS + s) * zrows, zrows)],
                              zsem).wait()


def _drain_1016(acc, out_ref, s, base):
    @pl.when(s < NS - 1)
    def _d0():
        pltpu.sync_copy(acc.at[pl.ds(s * 632, 632)],
                        out_ref.at[pl.ds(base + s * 632, 632)])

    @pl.when(s == NS - 1)
    def _d1():
        pltpu.sync_copy(acc.at[pl.ds((NS - 1) * 632, 520)],
                        out_ref.at[pl.ds(base + (NS - 1) * 632, 520)])


def _sc_mp_hidden(src_p, dst_p, h):
    @functools.partial(
        pl.kernel,
        out_type=jax.ShapeDtypeStruct((NC, N_HID, C), _f32),
        mesh=_mesh(),
        compiler_params=pltpu.CompilerParams(needs_layout_passes=False),
        scratch_types=[
            pltpu.VMEM_SHARED((HID_ACC, C), _f32),
            pltpu.VMEM((NCHUNK, CHUNK), _i32),
            pltpu.VMEM((NCHUNK, CHUNK), _i32),
            pltpu.VMEM((2, CHUNK, C), _f32),
            pltpu.VMEM((32, C), _f32),
            pltpu.SemaphoreType.DMA,
            pltpu.SemaphoreType.DMA,
            pltpu.SemaphoreType.DMA,
        ],
    )
    def sc_fn(src_r, dst_r, h_r, a_out, acc, srcv, dstv, rows, zbuf,
              g0, g1, zsem):
        c = lax.axis_index("c")
        s = lax.axis_index("s")
        wid = c * NS + s
        pltpu.sync_copy(src_r.at[wid], srcv)
        pltpu.sync_copy(dst_r.at[wid], dstv)
        zv = jnp.zeros((16,), _f32)

        @pl.loop(0, 32)
        def _zb(i):
            for t in range(C // 16):
                zbuf[i, pl.ds(t * 16, 16)] = zv

        _zero_rows(zbuf, acc, s, HID_ACC, 32, zsem)
        plsc.subcore_barrier()

        gsems = (g0, g1)
        pltpu.async_copy(h_r.at[srcv.at[0]], rows.at[0], g0)
        pltpu.async_copy(h_r.at[srcv.at[1]], rows.at[1], g1)

        @pl.loop(0, NCHUNK // 2)
        def _main(i):
            for b in range(2):
                j = i * 2 + b
                pltpu.make_async_copy(h_r.at[srcv.at[j]], rows.at[b],
                                      gsems[b]).wait()
                pltpu.sync_copy(rows.at[b], acc.at[dstv.at[j]], add=True)

                @pl.when(i * 2 + b + 2 < NCHUNK)
                def _nx(j=j, b=b):
                    pltpu.async_copy(h_r.at[srcv.at[j + 2]], rows.at[b],
                                     gsems[b])

        plsc.subcore_barrier()
        _drain_1016(acc, a_out.at[c], s, 0)

    return sc_fn(src_p, dst_p, h)


def _attr_accumulate(dstv, attrv, sacc, plane, lo, rng, trash):
    @pl.loop(0, NCHUNK)
    def _ch(j):
        for t in range(CHUNK // 16):
            d = dstv[j, pl.ds(t * 16, 16)]
            if rng is None:
                base = d
            else:
                m = (d >= lo) & (d < lo + rng)
                base = jnp.where(m, d - lo, trash)
            for k in range(EDIM):
                v = attrv[k, pl.ds(j * CHUNK + t * 16, 16)]
                plsc.addupdate_scatter(sacc, [base + k * plane], v)


def _zero_flat(sacc, n):
    zv = jnp.zeros((16,), _f32)

    @pl.loop(0, n // 16)
    def _zf(i):
        sacc[pl.ds(i * 16, 16)] = zv


def _sc_attrs_hidden(dstE, attrE, dstP, attrP):
    @functools.partial(
        pl.kernel,
        out_type=(jax.ShapeDtypeStruct((NW, (EDIM + 1) * HID_SPLANE), _f32),
                  jax.ShapeDtypeStruct((NW, (EDIM + 1) * HID_SPLANE), _f32)),
        mesh=_mesh(),
        compiler_params=pltpu.CompilerParams(needs_layout_passes=False),
        scratch_types=[
            pltpu.VMEM(((EDIM + 1) * HID_SPLANE,), _f32),
            pltpu.VMEM(((EDIM + 1) * HID_SPLANE,), _f32),
            pltpu.VMEM((NCHUNK, CHUNK), _i32),
            pltpu.VMEM((EDIM, EPW), _f32),
        ],
    )
    def sc_fn(dstE_r, attrE_r, dstP_r, attrP_r, sE_out, sP_out,
              saccE, saccP, dstv, attrv):
        c = lax.axis_index("c")
        s = lax.axis_index("s")
        wid = c * NS + s
        for dst_r, attr_r, sacc, out in ((dstE_r, attrE_r, saccE, sE_out),
                                         (dstP_r, attrP_r, saccP, sP_out)):
            pltpu.sync_copy(dst_r.at[wid], dstv)
            pltpu.sync_copy(attr_r.at[wid], attrv)
            _zero_flat(sacc, (EDIM + 1) * HID_SPLANE)
            _attr_accumulate(dstv, attrv, sacc, HID_SPLANE, 0, None, None)
            pltpu.sync_copy(sacc, out.at[wid])

    return sc_fn(dstE, attrE, dstP, attrP)


def _sc_attrs_dec(dstD, attrD):
    @functools.partial(
        pl.kernel,
        out_type=jax.ShapeDtypeStruct((NW, DECS_NP * (EDIM + 1) * DECS_PLANE), _f32),
        mesh=_mesh(),
        compiler_params=pltpu.CompilerParams(needs_layout_passes=False),
        scratch_types=[
            pltpu.VMEM(((EDIM + 1) * DECS_PLANE,), _f32),
            pltpu.VMEM((NCHUNK, CHUNK), _i32),
            pltpu.VMEM((EDIM, EPW), _f32),
        ],
    )
    def sc_fn(dst_r, attr_r, s_out, sacc, dstv, attrv):
        c = lax.axis_index("c")
        s = lax.axis_index("s")
        wid = c * NS + s
        pltpu.sync_copy(dst_r.at[wid], dstv)
        pltpu.sync_copy(attr_r.at[wid], attrv)
        trash = jnp.full((16,), DECS_RNG, _i32)
        for r in range(DECS_NP):
            _zero_flat(sacc, (EDIM + 1) * DECS_PLANE)
            _attr_accumulate(dstv, attrv, sacc, DECS_PLANE, r * DECS_RNG,
                             DECS_RNG, trash)
            pltpu.sync_copy(sacc, s_out.at[wid, pl.ds(r * (EDIM + 1) * DECS_PLANE, (EDIM + 1) * DECS_PLANE)])

    return sc_fn(dstD, attrD)


def _sc_mp_dec(pk_p, h):
    @functools.partial(
        pl.kernel,
        out_type=jax.ShapeDtypeStruct((NC, DEC_NP * DEC_RNG, C), _f32),
        mesh=_mesh(),
        compiler_params=pltpu.CompilerParams(needs_layout_passes=False),
        scratch_types=[
            pltpu.VMEM_SHARED((DEC_ACC, C), _f32),
            pltpu.VMEM((NCHUNK, CHUNK), _i32),
            pltpu.VMEM((NCHUNK, CHUNK), _i32),
            pltpu.VMEM((NCHUNK, CHUNK), _i32),
            pltpu.VMEM((CHUNK, C), _f32),
            pltpu.VMEM((64, C), _f32),
            pltpu.SemaphoreType.DMA,
            pltpu.SemaphoreType.DMA,
        ],
    )
    def sc_fn(pk_r, h_r, a_out,
              acc, pkv, scomp, dcomp, rows, zbuf, g0, zsem):
        c = lax.axis_index("c")
        s = lax.axis_index("s")
        wid = c * NS + s
        pltpu.sync_copy(pk_r.at[wid], pkv)
        zv = jnp.zeros((16,), _f32)
        z16i = jnp.zeros((16,), _i32)
        trash16 = jnp.full((16,), DEC_RNG, _i32)

        @pl.loop(0, 64)
        def _zb(i):
            for t in range(C // 16):
                zbuf[i, pl.ds(t * 16, 16)] = zv

        for r in range(DEC_NP):
            lo = r * DEC_RNG
            _zero_rows(zbuf, acc, s, DEC_ACC, 64, zsem)

            @pl.loop(0, NCHUNK)
            def _pf(j):
                for t in range(CHUNK // 16):
                    dcomp[j, pl.ds(t * 16, 16)] = trash16
                    scomp[j, pl.ds(t * 16, 16)] = z16i

            plsc.subcore_barrier()

            def cbody(j, cur):
                for t in range(CHUNK // 16):
                    w = pkv[j, pl.ds(t * 16, 16)]
                    d = lax.shift_right_logical(w, 16)
                    m = (d >= lo) & (d < lo + DEC_RNG)
                    dl = d - lo
                    pos = jnp.maximum(cur + plsc.cumsum(m.astype(_i32)) - 1, 0)
                    row = lax.shift_right_logical(pos, 7)
                    col = lax.bitwise_and(pos, 127)
                    sv = lax.bitwise_and(w, 0xFFFF)
                    plsc.store_scatter(dcomp, [row, col], dl, mask=m)
                    plsc.store_scatter(scomp, [row, col], sv, mask=m)
                    cur = cur + plsc.all_reduce_population_count(m)
                return cur

            cur = lax.fori_loop(0, NCHUNK, cbody, jnp.zeros((16,), _i32))
            nch = lax.shift_right_logical(jnp.max(cur) + (CHUNK - 1), 7)

            def gbody(j, carry):
                cp = pltpu.async_copy(h_r.at[scomp.at[j]], rows, g0)
                cp.wait()
                pltpu.sync_copy(rows, acc.at[dcomp.at[j]], add=True)
                return carry

            lax.fori_loop(0, nch, gbody, 0)
            plsc.subcore_barrier()
            _drain_1016(acc, a_out.at[c], s, lo)
            plsc.subcore_barrier()

    return sc_fn(pk_p, h)


def _dot(a, b):
    return jnp.dot(a, b, preferred_element_type=_f32)


def _embed_call(x0, x1, ga, w_src, wm):
    BLK = 1000

    def body(x0_r, x1_r, ga_r, w_r, wm_r, xd_r, he_r):
        xd = (_dot(x0_r[...], w_r[0:VARS])
              + _dot(x1_r[...], w_r[VARS:2 * VARS])
              + _dot(ga_r[...], w_r[2 * VARS:2 * VARS + ATTR]))
        xd_r[...] = xd
        he_r[...] = _dot(xd, wm_r[...])

    return pl.pallas_call(
        body,
        grid=(N_GRID // BLK,),
        in_specs=[
            pl.BlockSpec((BLK, VARS), lambda i: (i, 0)),
            pl.BlockSpec((BLK, VARS), lambda i: (i, 0)),
            pl.BlockSpec((BLK, ATTR), lambda i: (i, 0)),
            pl.BlockSpec((TIME * VARS + ATTR, C), lambda i: (0, 0)),
            pl.BlockSpec((C, C), lambda i: (0, 0)),
        ],
        out_specs=(pl.BlockSpec((BLK, C), lambda i: (i, 0)),
                   pl.BlockSpec((BLK, C), lambda i: (i, 0))),
        out_shape=(jax.ShapeDtypeStruct((N_GRID, C), _f32),
                   jax.ShapeDtypeStruct((N_GRID, C), _f32)),
    )(x0, x1, ga, w_src, wm)


def _s_term(t, s_blk, wewu):
    ssum = jnp.sum(s_blk, axis=0)
    for k in range(EDIM):
        t = t + ssum[k][:, None] * wewu[k][None, :]
    return t


def _post_enc_call(A, S, ha, we, wu, wdst, ws, wm0):
    BLK = 2048

    def body(a_r, s_r, ha_r, we_r, wu_r, wd_r, ws_r, wm_r, xh_r, h_r):
        t = _dot(a_r[0] + a_r[1], wu_r[...])
        t = _s_term(t, s_r[...], _dot(we_r[...], wu_r[...]))
        t = t + _dot(ha_r[...], _dot(wd_r[...], ws_r[...]))
        xh = jnp.maximum(t, 0.0)
        xh_r[...] = xh
        h_r[...] = _dot(xh, wm_r[...])

    return pl.pallas_call(
        body,
        grid=(pl.cdiv(N_HID, BLK),),
        in_specs=[
            pl.BlockSpec((NC, BLK, C), lambda i: (0, i, 0)),
            pl.BlockSpec((NW, EDIM + 1, BLK), lambda i: (0, 0, i)),
            pl.BlockSpec((BLK, ATTR), lambda i: (i, 0)),
            pl.BlockSpec((EDIM, C), lambda i: (0, 0)),
            pl.BlockSpec((C, C), lambda i: (0, 0)),
            pl.BlockSpec((ATTR, C), lambda i: (0, 0)),
            pl.BlockSpec((C, C), lambda i: (0, 0)),
            pl.BlockSpec((C, C), lambda i: (0, 0)),
        ],
        out_specs=(pl.BlockSpec((BLK, C), lambda i: (i, 0)),
                   pl.BlockSpec((BLK, C), lambda i: (i, 0))),
        out_shape=(jax.ShapeDtypeStruct((N_HID, C), _f32),
                   jax.ShapeDtypeStruct((N_HID, C), _f32)),
    )(A, S, ha, we, wu, wdst, ws, wm0)


def _post_proc_call(A, S, xh, we, wu, ws, wm_next):
    BLK = 2048

    def body(a_r, s_r, xh_r, we_r, wu_r, ws_r, wm_r, xo_r, h_r):
        xh = xh_r[...]
        t = _dot(a_r[0] + a_r[1], wu_r[...])
        t = _s_term(t, s_r[...], _dot(we_r[...], wu_r[...]))
        t = t + _dot(xh, ws_r[...])
        xo = xh + jnp.maximum(t, 0.0)
        xo_r[...] = xo
        h_r[...] = _dot(xo, wm_r[...])

    return pl.pallas_call(
        body,
        grid=(pl.cdiv(N_HID, BLK),),
        in_specs=[
            pl.BlockSpec((NC, BLK, C), lambda i: (0, i, 0)),
            pl.BlockSpec((NW, EDIM + 1, BLK), lambda i: (0, 0, i)),
            pl.BlockSpec((BLK, C), lambda i: (i, 0)),
            pl.BlockSpec((EDIM, C), lambda i: (0, 0)),
            pl.BlockSpec((C, C), lambda i: (0, 0)),
            pl.BlockSpec((C, C), lambda i: (0, 0)),
            pl.BlockSpec((C, C), lambda i: (0, 0)),
        ],
        out_specs=(pl.BlockSpec((BLK, C), lambda i: (i, 0)),
                   pl.BlockSpec((BLK, C), lambda i: (i, 0))),
        out_shape=(jax.ShapeDtypeStruct((N_HID, C), _f32),
                   jax.ShapeDtypeStruct((N_HID, C), _f32)),
    )(A, S, xh, we, wu, ws, wm_next)


def _final_call(A, S, xd, we, wu, ws, wout):
    BLK = 2048

    def body(a_r, s_r, xd_r, we_r, wu_r, ws_r, wo_r, o_r):
        t = _dot(a_r[0] + a_r[1], wu_r[...])
        t = _s_term(t, s_r[:, 0], _dot(we_r[...], wu_r[...]))
        t = t + _dot(xd_r[...], ws_r[...])
        o_r[...] = _dot(jnp.maximum(t, 0.0), wo_r[...])

    return pl.pallas_call(
        body,
        grid=(pl.cdiv(N_GRID, BLK),),
        in_specs=[
            pl.BlockSpec((NC, BLK, C), lambda i: (0, i, 0)),
            pl.BlockSpec((NW, 1, EDIM + 1, BLK), lambda i: (0, i // 5, 0, i % 5)),
            pl.BlockSpec((BLK, C), lambda i: (i, 0)),
            pl.BlockSpec((EDIM, C), lambda i: (0, 0)),
            pl.BlockSpec((C, C), lambda i: (0, 0)),
            pl.BlockSpec((C, C), lambda i: (0, 0)),
            pl.BlockSpec((C, VARS), lambda i: (0, 0)),
        ],
        out_specs=pl.BlockSpec((BLK, VARS), lambda i: (i, 0)),
        out_shape=jax.ShapeDtypeStruct((N_GRID, VARS), _f32),
    )(A, S, xd, we, wu, ws, wout)


def kernel(x, edge_index_enc, edge_attr_enc, edge_index_proc, edge_attr_proc,
           edge_index_dec, edge_attr_dec, grid_attrs, hidden_attrs,
           W_src_emb, W_dst_emb,
           enc_Wm, enc_We, enc_Wu, enc_Ws,
           proc_Wm, proc_We, proc_Wu, proc_Ws,
           dec_Wm, dec_We, dec_Wu, dec_Ws, W_out):
    x0 = x[0, 0, 0]
    x1 = x[0, 1, 0]
    srcE, dstE, attrE = _prep_edges(edge_index_enc, edge_attr_enc, N_HID)
    srcP, dstP, attrP = _prep_edges(edge_index_proc, edge_attr_proc, N_HID)
    srcD, dstD, attrD = _prep_edges(edge_index_dec, edge_attr_dec, N_GRID)
    pk = jnp.bitwise_or(
        jnp.left_shift(edge_index_dec[1].astype(_i32), 16),
        edge_index_dec[0].astype(_i32))
    pkD = jnp.concatenate(
        [pk, jnp.full((EPAD - E,), N_GRID << 16, _i32)]).reshape(NW, NCHUNK, CHUNK)

    x_data, h = _embed_call(x0, x1, grid_attrs, W_src_emb, enc_Wm)

    Se, Sp = _sc_attrs_hidden(dstE, attrE, dstP, attrP)
    Se = Se.reshape(NW, EDIM + 1, HID_SPLANE)
    Sp = Sp.reshape(NW, EDIM + 1, HID_SPLANE)
    Sd = _sc_attrs_dec(dstD, attrD).reshape(NW, DECS_NP, EDIM + 1, DECS_PLANE)

    A = _sc_mp_hidden(srcE, dstE, h)
    x_hid, h = _post_enc_call(A, Se, hidden_attrs, enc_We, enc_Wu,
                              W_dst_emb, enc_Ws, proc_Wm[0])

    for l in range(2):
        A = _sc_mp_hidden(srcP, dstP, h)
        wm_next = proc_Wm[1] if l == 0 else dec_Wm
        x_hid, h = _post_proc_call(A, Sp, x_hid, proc_We[l], proc_Wu[l],
                                   proc_Ws[l], wm_next)

    A = _sc_mp_dec(pkD, h)
    out = _final_call(A, Sd, x_data, dec_We, dec_Wu, dec_Ws, W_out)
    return out.reshape(1, 1, N_GRID, VARS)
